# baseline (speedup 1.0000x reference)
import functools

import jax
import jax.numpy as jnp
from jax import lax
from jax.experimental import pallas as pl
from jax.experimental.pallas import tpu as pltpu
from jax.experimental.pallas import tpu_sc as plsc

MEAN = 499.5
STD = 288.67
EPS = 1e-05
H = 128
NF = 13
VOCAB = 1001


def _idx_body(x_ref, idx_ref):
    x = x_ref[...]
    blk = x.shape[0]
    ii = lax.broadcasted_iota(jnp.int32, (NF, 2 * NF), 0)
    jj = lax.broadcasted_iota(jnp.int32, (NF, 2 * NF), 1)
    eselT = (jj == 2 * ii + 1).astype(jnp.float32)
    xselT = lax.dot_general(
        eselT,
        x,
        (((1,), (1,)), ((), ())),
        preferred_element_type=jnp.float32,
        precision=lax.Precision.HIGHEST,
    )
    idx_ref[...] = xselT.astype(jnp.int32) + VOCAB * lax.broadcasted_iota(
        jnp.int32, (NF, blk), 0
    )


def _idx_tc(x):
    B, A = x.shape
    BLK = 4096
    return pl.pallas_call(
        _idx_body,
        grid=(B // BLK,),
        in_specs=[pl.BlockSpec((BLK, A), lambda i: (i, 0))],
        out_specs=pl.BlockSpec((NF, BLK), lambda i: (0, i)),
        out_shape=jax.ShapeDtypeStruct((NF, B), jnp.int32),
    )(x)


def _combine_body(x_ref, linW_ref, linB_ref, cat_ref, out_ref):
    x = x_ref[...]
    s = jax.nn.sigmoid((x - MEAN) / (STD + EPS))
    w = linW_ref[...]
    w26 = jnp.stack([w, jnp.zeros_like(w)], axis=1).reshape(2 * NF, H)
    num = lax.dot_general(
        s, w26, (((1,), (0,)), ((), ())), preferred_element_type=jnp.float32
    )
    bias = jnp.sum(linB_ref[...], axis=0, keepdims=True)
    cat = cat_ref[...].reshape(x.shape[0], H).astype(jnp.float32)
    out_ref[...] = num + bias + cat


def _combine_tc(x, linW, linB, cat16):
    B, A = x.shape
    BLK = 2048
    return pl.pallas_call(
        _combine_body,
        grid=(B // BLK,),
        in_specs=[
            pl.BlockSpec((BLK, A), lambda i: (i, 0)),
            pl.BlockSpec((NF, H), lambda i: (0, 0)),
            pl.BlockSpec((NF, H), lambda i: (0, 0)),
            pl.BlockSpec((BLK * H,), lambda i: (i,)),
        ],
        out_specs=pl.BlockSpec((BLK, H), lambda i: (i, 0)),
        out_shape=jax.ShapeDtypeStruct((B, H), jnp.float32),
    )(x, linW, linB, cat16)


def _make_sc_cat(B):
    NW = 32
    RPW = B // NW
    NCHUNK = RPW // 128

    mesh = plsc.VectorSubcoreMesh(core_axis_name="c", subcore_axis_name="s")

    @functools.partial(
        pl.kernel,
        out_type=jax.ShapeDtypeStruct((B, H), jnp.bfloat16),
        mesh=mesh,
        compiler_params=pltpu.CompilerParams(use_tc_tiling_on_sc=False),
        scratch_types=[
            pltpu.VMEM((NF, NCHUNK, 128), jnp.int32),
            pltpu.VMEM((RPW, H), jnp.bfloat16),
            pltpu.SemaphoreType.DMA,
            pltpu.SemaphoreType.DMA,
            pltpu.SemaphoreType.DMA,
        ],
    )
    def sc_cat(idx_hbm, tab_hbm, out_hbm, idx_v, acc_v, isem, gsem, osem):
        tab2d = tab_hbm
        wid = lax.axis_index("s") * 2 + lax.axis_index("c")
        base = wid * RPW
        idx_copies = [
            [
                pltpu.async_copy(
                    idx_hbm.at[pl.ds(j * B + base + 128 * c, 128)],
                    idx_v.at[j, c],
                    isem,
                )
                for j in range(NF)
            ]
            for c in range(NCHUNK)
        ]
        first = []
        for c in range(NCHUNK):
            idx_copies[c][0].wait()
            first.append(
                pltpu.async_copy(
                    tab2d.at[idx_v.at[0, c]],
                    acc_v.at[pl.ds(c * 128, 128)],
                    gsem,
                )
            )
        rest = []
        for c in range(NCHUNK):
            first[c].wait()
            for j in range(1, NF):
                idx_copies[c][j].wait()
            rest.append(
                [
                    pltpu.async_copy(
                        tab2d.at[idx_v.at[j, c]],
                        acc_v.at[pl.ds(c * 128, 128)],
                        gsem,
                        add=True,
                    )
                    for j in range(1, NF)
                ]
            )
        out_copies = []
        for c in range(NCHUNK):
            for cp in rest[c]:
                cp.wait()
            out_copies.append(
                pltpu.async_copy(
                    acc_v.at[pl.ds(c * 128, 128)],
                    out_hbm.at[pl.ds(base + c * 128, 128)],
                    osem,
                )
            )
        for cp in out_copies:
            cp.wait()

    return sc_cat


def kernel(x, linW, linB, tables):
    B, A = x.shape
    idxT = _idx_tc(x)
    tab16 = tables.astype(jnp.bfloat16).reshape(NF * VOCAB, H)
    cat16 = _make_sc_cat(B)(idxT.reshape(NF * B), tab16)
    return _combine_tc(x, linW, linB, cat16.reshape(B * H))


# speedup vs baseline: 1.3184x; 1.3184x over previous
import functools

import jax
import jax.numpy as jnp
from jax import lax
from jax.experimental import pallas as pl
from jax.experimental.pallas import tpu as pltpu
from jax.experimental.pallas import tpu_sc as plsc

MEAN = 499.5
STD = 288.67
EPS = 1e-05
H = 128
NF = 13
VOCAB = 1001


def _tc_body(x_ref, linW_ref, linB_ref, num_ref, idx_ref):
    x = x_ref[...]
    blk = x.shape[0]
    s = jax.nn.sigmoid((x - MEAN) / (STD + EPS))
    w = linW_ref[...]
    w26 = jnp.stack([w, jnp.zeros_like(w)], axis=1).reshape(2 * NF, H)
    num = lax.dot_general(
        s, w26, (((1,), (0,)), ((), ())), preferred_element_type=jnp.float32
    )
    num_ref[...] = num + jnp.sum(linB_ref[...], axis=0, keepdims=True)
    ii = lax.broadcasted_iota(jnp.int32, (NF, 2 * NF), 0)
    jj = lax.broadcasted_iota(jnp.int32, (NF, 2 * NF), 1)
    eselT = (jj == 2 * ii + 1).astype(jnp.float32)
    xselT = lax.dot_general(
        eselT,
        x,
        (((1,), (1,)), ((), ())),
        preferred_element_type=jnp.float32,
        precision=lax.Precision.HIGHEST,
    )
    del blk
    idx_ref[...] = xselT.astype(jnp.int32)


def _numeric_and_idx_tc(x, linW, linB):
    B, A = x.shape
    BLK = 2048
    return pl.pallas_call(
        _tc_body,
        grid=(B // BLK,),
        in_specs=[
            pl.BlockSpec((BLK, A), lambda i: (i, 0)),
            pl.BlockSpec((NF, H), lambda i: (0, 0)),
            pl.BlockSpec((NF, H), lambda i: (0, 0)),
        ],
        out_specs=[
            pl.BlockSpec((BLK, H), lambda i: (i, 0)),
            pl.BlockSpec((NF, BLK), lambda i: (0, i)),
        ],
        out_shape=[
            jax.ShapeDtypeStruct((B, H), jnp.float32),
            jax.ShapeDtypeStruct((NF, B), jnp.int32),
        ],
    )(x, linW, linB)


def _make_sc_embed(B):
    NW = 32
    RPW = B // NW
    NCHUNK = RPW // 128

    mesh = plsc.VectorSubcoreMesh(core_axis_name="c", subcore_axis_name="s")

    @functools.partial(
        pl.kernel,
        out_type=jax.ShapeDtypeStruct((B, H), jnp.float32),
        mesh=mesh,
        scratch_types=[
            pltpu.VMEM((NF, NCHUNK, 128), jnp.int32),
            pltpu.VMEM((RPW, H), jnp.float32),
            pltpu.SemaphoreType.DMA,
            pltpu.SemaphoreType.DMA,
            pltpu.SemaphoreType.DMA,
            pltpu.SemaphoreType.DMA,
        ],
    )
    def sc_embed(
        idx_hbm, tab_hbm, num_hbm, out_hbm, idx_v, acc_v, isem, gsem, nsem, osem
    ):
        wid = lax.axis_index("s") * 2 + lax.axis_index("c")
        base = wid * RPW
        num_copies = [
            pltpu.async_copy(
                num_hbm.at[pl.ds(base + 128 * c, 128)],
                acc_v.at[pl.ds(128 * c, 128)],
                nsem,
            )
            for c in range(NCHUNK)
        ]
        idx_copies = [
            [
                pltpu.async_copy(
                    idx_hbm.at[pl.ds(j * B + base + 128 * c, 128)],
                    idx_v.at[j, c],
                    isem,
                )
                for j in range(NF)
            ]
            for c in range(NCHUNK)
        ]
        gathers = []
        for c in range(NCHUNK):
            num_copies[c].wait()
            for cp in idx_copies[c]:
                cp.wait()
            gathers.append(
                [
                    pltpu.async_copy(
                        tab_hbm.at[j].at[idx_v.at[j, c]],
                        acc_v.at[pl.ds(c * 128, 128)],
                        gsem,
                        add=True,
                    )
                    for j in range(NF)
                ]
            )
        out_copies = []
        for c in range(NCHUNK):
            for cp in gathers[c]:
                cp.wait()
            out_copies.append(
                pltpu.async_copy(
                    acc_v.at[pl.ds(c * 128, 128)],
                    out_hbm.at[pl.ds(base + c * 128, 128)],
                    osem,
                )
            )
        for cp in out_copies:
            cp.wait()

    return sc_embed


def kernel(x, linW, linB, tables):
    B, A = x.shape
    numeric, idxT = _numeric_and_idx_tc(x, linW, linB)
    return _make_sc_embed(B)(idxT.reshape(NF * B), tables, numeric)


# speedup vs baseline: 1.3323x; 1.0106x over previous
import functools

import jax
import jax.numpy as jnp
from jax import lax
from jax.experimental import pallas as pl
from jax.experimental.pallas import tpu as pltpu
from jax.experimental.pallas import tpu_sc as plsc

MEAN = 499.5
STD = 288.67
EPS = 1e-05
H = 128
NF = 13
VOCAB = 1001


def _tc_body(x_ref, linW_ref, linB_ref, num_ref, idx_ref):
    x = x_ref[...]
    blk = x.shape[0]
    s = jax.nn.sigmoid((x - MEAN) / (STD + EPS))
    w = linW_ref[...]
    w26 = jnp.stack([w, jnp.zeros_like(w)], axis=1).reshape(2 * NF, H)
    num = lax.dot_general(
        s, w26, (((1,), (0,)), ((), ())), preferred_element_type=jnp.float32
    )
    num_ref[...] = num + jnp.sum(linB_ref[...], axis=0, keepdims=True)
    ii = lax.broadcasted_iota(jnp.int32, (NF, 2 * NF), 0)
    jj = lax.broadcasted_iota(jnp.int32, (NF, 2 * NF), 1)
    eselT = (jj == 2 * ii + 1).astype(jnp.float32)
    xselT = lax.dot_general(
        eselT,
        x,
        (((1,), (1,)), ((), ())),
        preferred_element_type=jnp.float32,
        precision=lax.Precision.HIGHEST,
    )
    del blk
    idx_ref[...] = xselT.astype(jnp.int32)


def _numeric_and_idx_tc(x, linW, linB):
    B, A = x.shape
    BLK = 4096
    return pl.pallas_call(
        _tc_body,
        grid=(B // BLK,),
        in_specs=[
            pl.BlockSpec((BLK, A), lambda i: (i, 0)),
            pl.BlockSpec((NF, H), lambda i: (0, 0)),
            pl.BlockSpec((NF, H), lambda i: (0, 0)),
        ],
        out_specs=[
            pl.BlockSpec((BLK, H), lambda i: (i, 0)),
            pl.BlockSpec((NF, BLK), lambda i: (0, i)),
        ],
        out_shape=[
            jax.ShapeDtypeStruct((B, H), jnp.float32),
            jax.ShapeDtypeStruct((NF, B), jnp.int32),
        ],
    )(x, linW, linB)


def _make_sc_embed(B):
    NW = 32
    RPW = B // NW
    NCHUNK = RPW // 128

    mesh = plsc.VectorSubcoreMesh(core_axis_name="c", subcore_axis_name="s")

    @functools.partial(
        pl.kernel,
        out_type=jax.ShapeDtypeStruct((B, H), jnp.float32),
        mesh=mesh,
        scratch_types=[
            pltpu.VMEM((NF, NCHUNK, 128), jnp.int32),
            pltpu.VMEM((RPW, H), jnp.float32),
            pltpu.SemaphoreType.DMA,
            pltpu.SemaphoreType.DMA,
            pltpu.SemaphoreType.DMA,
            pltpu.SemaphoreType.DMA,
        ],
    )
    def sc_embed(
        idx_hbm, tab_hbm, num_hbm, out_hbm, idx_v, acc_v, isem, gsem, nsem, osem
    ):
        wid = lax.axis_index("s") * 2 + lax.axis_index("c")
        base = wid * RPW
        num_copies = [
            pltpu.async_copy(
                num_hbm.at[pl.ds(base + 128 * c, 128)],
                acc_v.at[pl.ds(128 * c, 128)],
                nsem,
            )
            for c in range(NCHUNK)
        ]
        idx_copies = [
            [
                pltpu.async_copy(
                    idx_hbm.at[pl.ds(j * B + base + 128 * c, 128)],
                    idx_v.at[j, c],
                    isem,
                )
                for j in range(NF)
            ]
            for c in range(NCHUNK)
        ]
        gathers = []
        for c in range(NCHUNK):
            num_copies[c].wait()
            for cp in idx_copies[c]:
                cp.wait()
            gathers.append(
                [
                    pltpu.async_copy(
                        tab_hbm.at[j].at[idx_v.at[j, c]],
                        acc_v.at[pl.ds(c * 128, 128)],
                        gsem,
                        add=True,
                    )
                    for j in range(NF)
                ]
            )
        out_copies = []
        for c in range(NCHUNK):
            for cp in gathers[c]:
                cp.wait()
            out_copies.append(
                pltpu.async_copy(
                    acc_v.at[pl.ds(c * 128, 128)],
                    out_hbm.at[pl.ds(base + c * 128, 128)],
                    osem,
                )
            )
        for cp in out_copies:
            cp.wait()

    return sc_embed


def kernel(x, linW, linB, tables):
    B, A = x.shape
    numeric, idxT = _numeric_and_idx_tc(x, linW, linB)
    return _make_sc_embed(B)(idxT.reshape(NF * B), tables, numeric)
